# Initial kernel scaffold; baseline (speedup 1.0000x reference)
import jax
import jax.numpy as jnp
from jax.experimental import pallas as pl


def kernel(fc_log, genotypes, expression_obs, variantxgene_to_gene, local_variant_to_local_variantxgene_selector, variantxgene_to_local_gene, lib, baseline_log, dispersion_log):
    raise NotImplementedError("write your pallas kernel here")


# speedup vs baseline: 3.2139x; 3.2139x over previous
import jax
import jax.numpy as jnp
from jax import lax
from jax.experimental import pallas as pl

N_DONORS = 64
N_CLUSTERS = 32
N_GENES = 2000
N_VARIANTS = 5000
N_VXG = 10000
EPS = 1e-8
_HALF_LOG_2PI = 0.9189385332046727

_BLK = 512


def _lgamma_pos(x):
    p = x * (x + 1.0) * (x + 2.0) * (x + 3.0) * (x + 4.0) * (x + 5.0) * (x + 6.0) * (x + 7.0)
    small = x < 8.0
    z = jnp.where(small, x + 8.0, x)
    zi = 1.0 / z
    zi2 = zi * zi
    ser = zi * (0.08333333333333333 + zi2 * (-0.002777777777777778
                                             + zi2 * 0.0007936507936507937))
    st = (z - 0.5) * jnp.log(z) - z + _HALF_LOG_2PI + ser
    return jnp.where(small, st - jnp.log(p), st)


def _prep_body(baseline_ref, dispersion_ref, m_ref):
    disp = jnp.minimum(jnp.exp(dispersion_ref[...]), 20.0)
    tc = 1.0 / disp
    m_ref[0:N_CLUSTERS, :] = baseline_ref[...]
    m_ref[N_CLUSTERS:2 * N_CLUSTERS, :] = tc
    m_ref[2 * N_CLUSTERS:3 * N_CLUSTERS, :] = jnp.log(tc + EPS)
    m_ref[3 * N_CLUSTERS:4 * N_CLUSTERS, :] = _lgamma_pos(tc)


def _main_body(gidx_ref, sidx_ref, lidx_ref, fc_ref, geno_ref, obs_ref,
               lib_ref, m_ref, expressed_ref, elbo_ref):
    B = fc_ref.shape[-1]
    gidx = gidx_ref[0]
    sidx = sidx_ref[0]
    lidx = lidx_ref[0]

    iota_gene = lax.broadcasted_iota(jnp.int32, (N_GENES, B), 0)
    iota_var = lax.broadcasted_iota(jnp.int32, (N_VARIANTS, B), 0)
    oh_gene = (iota_gene == gidx).astype(jnp.float32)
    oh_sel = (iota_var == sidx).astype(jnp.bfloat16)
    oh_lg = (iota_gene == lidx).astype(jnp.bfloat16)

    dn = (((1,), (0,)), ((), ()))
    mg = lax.dot_general(m_ref[...], oh_gene, dn,
                         precision=lax.Precision.HIGHEST,
                         preferred_element_type=jnp.float32)
    g = lax.dot_general(geno_ref[...], oh_sel, dn,
                        preferred_element_type=jnp.float32)
    value = lax.dot_general(obs_ref[...], oh_lg, dn,
                            preferred_element_type=jnp.float32)
    value = value.reshape(N_DONORS, N_CLUSTERS, B)

    baseline_g = mg[0:N_CLUSTERS]
    tc = mg[N_CLUSTERS:2 * N_CLUSTERS]
    l1 = mg[2 * N_CLUSTERS:3 * N_CLUSTERS]
    g0 = mg[3 * N_CLUSTERS:4 * N_CLUSTERS]

    el = baseline_g[None, :, :] + g[:, None, :] * fc_ref[...][None, :, :]
    expressed = jnp.exp(el) * lib_ref[...][:, :, None]
    expressed_ref[...] = expressed

    logits = jnp.log(expressed + EPS) - l1[None, :, :]
    sp = jnp.maximum(logits, 0.0) + jnp.log(1.0 + jnp.exp(-jnp.abs(logits)))
    tcv = tc[None, :, :] + value
    elbo = (tcv * sp - value * logits
            - _lgamma_pos(tcv) + _lgamma_pos(1.0 + value) + g0[None, :, :])
    elbo_ref[...] = elbo


def kernel(fc_log, genotypes, expression_obs, variantxgene_to_gene,
           local_variant_to_local_variantxgene_selector, variantxgene_to_local_gene,
           lib, baseline_log, dispersion_log):
    nblk = (N_VXG + _BLK - 1) // _BLK
    pad = nblk * _BLK - N_VXG

    m = pl.pallas_call(
        _prep_body,
        out_shape=jax.ShapeDtypeStruct((4 * N_CLUSTERS, N_GENES), jnp.float32),
    )(baseline_log, dispersion_log)

    def _idx(a):
        a = jnp.pad(a.astype(jnp.int32), (0, pad))
        return a.reshape(nblk, 1, _BLK)

    gidx = _idx(variantxgene_to_gene)
    sidx = _idx(local_variant_to_local_variantxgene_selector)
    lidx = _idx(variantxgene_to_local_gene)
    geno_bf = genotypes.astype(jnp.bfloat16)
    obs_bf = expression_obs.reshape(N_DONORS * N_CLUSTERS, N_GENES).astype(jnp.bfloat16)

    grid = (nblk,)
    out_specs = [
        pl.BlockSpec((N_DONORS, N_CLUSTERS, _BLK), lambda j: (0, 0, j)),
        pl.BlockSpec((N_DONORS, N_CLUSTERS, _BLK), lambda j: (0, 0, j)),
    ]
    in_specs = [
        pl.BlockSpec((1, 1, _BLK), lambda j: (j, 0, 0)),
        pl.BlockSpec((1, 1, _BLK), lambda j: (j, 0, 0)),
        pl.BlockSpec((1, 1, _BLK), lambda j: (j, 0, 0)),
        pl.BlockSpec((N_CLUSTERS, _BLK), lambda j: (0, j)),
        pl.BlockSpec((N_DONORS, N_VARIANTS), lambda j: (0, 0)),
        pl.BlockSpec((N_DONORS * N_CLUSTERS, N_GENES), lambda j: (0, 0)),
        pl.BlockSpec((N_DONORS, N_CLUSTERS), lambda j: (0, 0)),
        pl.BlockSpec((4 * N_CLUSTERS, N_GENES), lambda j: (0, 0)),
    ]
    expressed, elbo = pl.pallas_call(
        _main_body,
        grid=grid,
        in_specs=in_specs,
        out_specs=out_specs,
        out_shape=[
            jax.ShapeDtypeStruct((N_DONORS, N_CLUSTERS, N_VXG), jnp.float32),
            jax.ShapeDtypeStruct((N_DONORS, N_CLUSTERS, N_VXG), jnp.float32),
        ],
    )(gidx, sidx, lidx, fc_log, geno_bf, obs_bf, lib, m)
    return expressed, elbo


# speedup vs baseline: 4.1019x; 1.2763x over previous
import jax
import jax.numpy as jnp
from jax import lax
from jax.experimental import pallas as pl

N_DONORS = 64
N_CLUSTERS = 32
N_GENES = 2000
N_VARIANTS = 5000
N_VXG = 10000
EPS = 1e-8
_HALF_LOG_2PI = 0.9189385332046727

_BLK = 512


def _lgamma_pos(x):
    q = x * x + 3.0 * x
    p = q * (q + 2.0)
    small = x < 4.0
    z = jnp.where(small, x + 4.0, x)
    zi = 1.0 / z
    zi2 = zi * zi
    ser = zi * (0.08333333333333333 + zi2 * (-0.002777777777777778
                                             + zi2 * 0.0007936507936507937))
    st = (z - 0.5) * jnp.log(z) - z + _HALF_LOG_2PI + ser
    return jnp.where(small, st - jnp.log(p), st)


def _stirling(z):
    zi = 1.0 / z
    zi2 = zi * zi
    ser = zi * (0.08333333333333333 + zi2 * (-0.002777777777777778
                                             + zi2 * 0.0007936507936507937))
    return (z - 0.5) * jnp.log(z) - z + ser


def _lgamma_diff(xa, xb):
    qa = xa * xa + 3.0 * xa
    pa = qa * (qa + 2.0)
    small_a = xa < 4.0
    za = jnp.where(small_a, xa + 4.0, xa)
    qb = xb * xb + 3.0 * xb
    pb = qb * (qb + 2.0)
    small_b = xb < 4.0
    zb = jnp.where(small_b, xb + 4.0, xb)
    num = jnp.where(small_a, pa, 1.0)
    den = jnp.where(small_b, pb, 1.0)
    return _stirling(zb) - _stirling(za) + jnp.log(num / den)


def _prep_body(baseline_ref, dispersion_ref, m_ref):
    disp = jnp.minimum(jnp.exp(dispersion_ref[...]), 20.0)
    tc = 1.0 / disp
    m_ref[0:N_CLUSTERS, :] = baseline_ref[...]
    m_ref[N_CLUSTERS:2 * N_CLUSTERS, :] = tc
    m_ref[2 * N_CLUSTERS:3 * N_CLUSTERS, :] = jnp.log(tc + EPS)
    m_ref[3 * N_CLUSTERS:4 * N_CLUSTERS, :] = _lgamma_pos(tc)


def _main_body(gidx_ref, sidx_ref, lidx_ref, fc_ref, geno_ref, obs_ref,
               lib_ref, mhi_ref, mlo_ref, expressed_ref, elbo_ref):
    B = fc_ref.shape[-1]
    gidx = gidx_ref[0]
    sidx = sidx_ref[0]
    lidx = lidx_ref[0]

    iota_gene = lax.broadcasted_iota(jnp.int32, (N_GENES, B), 0)
    iota_var = lax.broadcasted_iota(jnp.int32, (N_VARIANTS, B), 0)
    oh_gene = (iota_gene == gidx).astype(jnp.bfloat16)
    oh_sel = (iota_var == sidx).astype(jnp.bfloat16)
    oh_lg = (iota_gene == lidx).astype(jnp.bfloat16)

    dn = (((1,), (0,)), ((), ()))
    mg = (lax.dot_general(mhi_ref[...], oh_gene, dn,
                          preferred_element_type=jnp.float32)
          + lax.dot_general(mlo_ref[...], oh_gene, dn,
                            preferred_element_type=jnp.float32))
    g = lax.dot_general(geno_ref[...], oh_sel, dn,
                        preferred_element_type=jnp.float32)
    value = lax.dot_general(obs_ref[...], oh_lg, dn,
                            preferred_element_type=jnp.float32)
    value = value.reshape(N_DONORS, N_CLUSTERS, B)

    baseline_g = mg[0:N_CLUSTERS]
    tc = mg[N_CLUSTERS:2 * N_CLUSTERS]
    l1 = mg[2 * N_CLUSTERS:3 * N_CLUSTERS]
    g0 = mg[3 * N_CLUSTERS:4 * N_CLUSTERS]

    el = baseline_g[None, :, :] + g[:, None, :] * fc_ref[...][None, :, :]
    expressed = jnp.exp(el) * lib_ref[...][:, :, None]
    expressed_ref[...] = expressed

    logits = jnp.log(expressed + EPS) - l1[None, :, :]
    sp = jnp.maximum(logits, 0.0) + jnp.log(1.0 + jnp.exp(-jnp.abs(logits)))
    tcv = tc[None, :, :] + value
    elbo = (tcv * sp - value * logits
            + _lgamma_diff(tcv, 1.0 + value) + g0[None, :, :])
    elbo_ref[...] = elbo


def kernel(fc_log, genotypes, expression_obs, variantxgene_to_gene,
           local_variant_to_local_variantxgene_selector, variantxgene_to_local_gene,
           lib, baseline_log, dispersion_log):
    nblk = (N_VXG + _BLK - 1) // _BLK
    pad = nblk * _BLK - N_VXG

    m = pl.pallas_call(
        _prep_body,
        out_shape=jax.ShapeDtypeStruct((4 * N_CLUSTERS, N_GENES), jnp.float32),
    )(baseline_log, dispersion_log)

    def _idx(a):
        a = jnp.pad(a.astype(jnp.int32), (0, pad))
        return a.reshape(nblk, 1, _BLK)

    gidx = _idx(variantxgene_to_gene)
    sidx = _idx(local_variant_to_local_variantxgene_selector)
    lidx = _idx(variantxgene_to_local_gene)
    m_hi = m.astype(jnp.bfloat16)
    m_lo = (m - m_hi.astype(jnp.float32)).astype(jnp.bfloat16)
    geno_bf = genotypes.astype(jnp.bfloat16)
    obs_bf = expression_obs.reshape(N_DONORS * N_CLUSTERS, N_GENES).astype(jnp.bfloat16)

    grid = (nblk,)
    out_specs = [
        pl.BlockSpec((N_DONORS, N_CLUSTERS, _BLK), lambda j: (0, 0, j)),
        pl.BlockSpec((N_DONORS, N_CLUSTERS, _BLK), lambda j: (0, 0, j)),
    ]
    in_specs = [
        pl.BlockSpec((1, 1, _BLK), lambda j: (j, 0, 0)),
        pl.BlockSpec((1, 1, _BLK), lambda j: (j, 0, 0)),
        pl.BlockSpec((1, 1, _BLK), lambda j: (j, 0, 0)),
        pl.BlockSpec((N_CLUSTERS, _BLK), lambda j: (0, j)),
        pl.BlockSpec((N_DONORS, N_VARIANTS), lambda j: (0, 0)),
        pl.BlockSpec((N_DONORS * N_CLUSTERS, N_GENES), lambda j: (0, 0)),
        pl.BlockSpec((N_DONORS, N_CLUSTERS), lambda j: (0, 0)),
        pl.BlockSpec((4 * N_CLUSTERS, N_GENES), lambda j: (0, 0)),
        pl.BlockSpec((4 * N_CLUSTERS, N_GENES), lambda j: (0, 0)),
    ]
    expressed, elbo = pl.pallas_call(
        _main_body,
        grid=grid,
        in_specs=in_specs,
        out_specs=out_specs,
        out_shape=[
            jax.ShapeDtypeStruct((N_DONORS, N_CLUSTERS, N_VXG), jnp.float32),
            jax.ShapeDtypeStruct((N_DONORS, N_CLUSTERS, N_VXG), jnp.float32),
        ],
    )(gidx, sidx, lidx, fc_log, geno_bf, obs_bf, lib, m_hi, m_lo)
    return expressed, elbo


# speedup vs baseline: 4.1224x; 1.0050x over previous
import jax
import jax.numpy as jnp
from jax import lax
from jax.experimental import pallas as pl

N_DONORS = 64
N_CLUSTERS = 32
N_GENES = 2000
N_VARIANTS = 5000
N_VXG = 10000
EPS = 1e-8
_HALF_LOG_2PI = 0.9189385332046727

_BLK = 512


def _lgamma_pos(x):
    q = x * x + 3.0 * x
    p = q * (q + 2.0)
    small = x < 4.0
    z = jnp.where(small, x + 4.0, x)
    zi = 1.0 / z
    zi2 = zi * zi
    ser = zi * (0.08333333333333333 + zi2 * (-0.002777777777777778
                                             + zi2 * 0.0007936507936507937))
    st = (z - 0.5) * jnp.log(z) - z + _HALF_LOG_2PI + ser
    return jnp.where(small, st - jnp.log(p), st)


def _stirling(z):
    zi = 1.0 / z
    zi2 = zi * zi
    ser = zi * (0.08333333333333333 + zi2 * (-0.002777777777777778
                                             + zi2 * 0.0007936507936507937))
    return (z - 0.5) * jnp.log(z) - z + ser


def _lgamma_diff(xa, xb):
    qa = xa * xa + 3.0 * xa
    pa = qa * (qa + 2.0)
    small_a = xa < 4.0
    za = jnp.where(small_a, xa + 4.0, xa)
    qb = xb * xb + 3.0 * xb
    pb = qb * (qb + 2.0)
    small_b = xb < 4.0
    zb = jnp.where(small_b, xb + 4.0, xb)
    num = jnp.where(small_a, pa, 1.0)
    den = jnp.where(small_b, pb, 1.0)
    return _stirling(zb) - _stirling(za) + jnp.log(num / den)


def _prep_body(baseline_ref, dispersion_ref, m_ref):
    disp = jnp.minimum(jnp.exp(dispersion_ref[...]), 20.0)
    tc = 1.0 / disp
    m_ref[0:N_CLUSTERS, :] = baseline_ref[...]
    m_ref[N_CLUSTERS:2 * N_CLUSTERS, :] = tc
    m_ref[2 * N_CLUSTERS:3 * N_CLUSTERS, :] = jnp.log(tc + EPS)
    m_ref[3 * N_CLUSTERS:4 * N_CLUSTERS, :] = _lgamma_pos(tc)


def _main_body(gidx_ref, sidx_ref, lidx_ref, fc_ref, geno_ref, obs_ref,
               lib_ref, m2_ref, expressed_ref, elbo_ref):
    B = fc_ref.shape[-1]
    gidx = gidx_ref[0]
    sidx = sidx_ref[0]
    lidx = lidx_ref[0]

    iota_gene = lax.broadcasted_iota(jnp.int32, (N_GENES, B), 0)
    iota_var = lax.broadcasted_iota(jnp.int32, (N_VARIANTS, B), 0)
    oh_gene = (iota_gene == gidx).astype(jnp.bfloat16)
    oh_sel = (iota_var == sidx).astype(jnp.bfloat16)
    oh_lg = (iota_gene == lidx).astype(jnp.bfloat16)

    dn = (((1,), (0,)), ((), ()))
    mg2 = lax.dot_general(m2_ref[...], oh_gene, dn,
                          preferred_element_type=jnp.float32)
    mg = mg2[0:4 * N_CLUSTERS] + mg2[4 * N_CLUSTERS:]
    g = lax.dot_general(geno_ref[...], oh_sel, dn,
                        preferred_element_type=jnp.float32)
    value = lax.dot_general(obs_ref[...], oh_lg, dn,
                            preferred_element_type=jnp.float32)
    value = value.reshape(N_DONORS, N_CLUSTERS, B)

    baseline_g = mg[0:N_CLUSTERS]
    tc = mg[N_CLUSTERS:2 * N_CLUSTERS]
    l1 = mg[2 * N_CLUSTERS:3 * N_CLUSTERS]
    g0 = mg[3 * N_CLUSTERS:4 * N_CLUSTERS]

    el = baseline_g[None, :, :] + g[:, None, :] * fc_ref[...][None, :, :]
    expressed = jnp.exp(el) * lib_ref[...][:, :, None]
    expressed_ref[...] = expressed

    logits = jnp.log(expressed + EPS) - l1[None, :, :]
    sp = jnp.maximum(logits, 0.0) + jnp.log(1.0 + jnp.exp(-jnp.abs(logits)))
    tcv = tc[None, :, :] + value
    elbo = (tcv * sp - value * logits
            + _lgamma_diff(tcv, 1.0 + value) + g0[None, :, :])
    elbo_ref[...] = elbo


def kernel(fc_log, genotypes, expression_obs, variantxgene_to_gene,
           local_variant_to_local_variantxgene_selector, variantxgene_to_local_gene,
           lib, baseline_log, dispersion_log):
    nblk = (N_VXG + _BLK - 1) // _BLK
    pad = nblk * _BLK - N_VXG

    m = pl.pallas_call(
        _prep_body,
        out_shape=jax.ShapeDtypeStruct((4 * N_CLUSTERS, N_GENES), jnp.float32),
    )(baseline_log, dispersion_log)

    def _idx(a):
        a = jnp.pad(a.astype(jnp.int32), (0, pad))
        return a.reshape(nblk, 1, _BLK)

    gidx = _idx(variantxgene_to_gene)
    sidx = _idx(local_variant_to_local_variantxgene_selector)
    lidx = _idx(variantxgene_to_local_gene)
    m_hi = m.astype(jnp.bfloat16)
    m_lo = (m - m_hi.astype(jnp.float32)).astype(jnp.bfloat16)
    m2 = jnp.concatenate([m_hi, m_lo], axis=0)
    geno_bf = genotypes.astype(jnp.bfloat16)
    obs_bf = expression_obs.reshape(N_DONORS * N_CLUSTERS, N_GENES).astype(jnp.bfloat16)

    grid = (nblk,)
    out_specs = [
        pl.BlockSpec((N_DONORS, N_CLUSTERS, _BLK), lambda j: (0, 0, j)),
        pl.BlockSpec((N_DONORS, N_CLUSTERS, _BLK), lambda j: (0, 0, j)),
    ]
    in_specs = [
        pl.BlockSpec((1, 1, _BLK), lambda j: (j, 0, 0)),
        pl.BlockSpec((1, 1, _BLK), lambda j: (j, 0, 0)),
        pl.BlockSpec((1, 1, _BLK), lambda j: (j, 0, 0)),
        pl.BlockSpec((N_CLUSTERS, _BLK), lambda j: (0, j)),
        pl.BlockSpec((N_DONORS, N_VARIANTS), lambda j: (0, 0)),
        pl.BlockSpec((N_DONORS * N_CLUSTERS, N_GENES), lambda j: (0, 0)),
        pl.BlockSpec((N_DONORS, N_CLUSTERS), lambda j: (0, 0)),
        pl.BlockSpec((8 * N_CLUSTERS, N_GENES), lambda j: (0, 0)),
    ]
    expressed, elbo = pl.pallas_call(
        _main_body,
        grid=grid,
        in_specs=in_specs,
        out_specs=out_specs,
        out_shape=[
            jax.ShapeDtypeStruct((N_DONORS, N_CLUSTERS, N_VXG), jnp.float32),
            jax.ShapeDtypeStruct((N_DONORS, N_CLUSTERS, N_VXG), jnp.float32),
        ],
    )(gidx, sidx, lidx, fc_log, geno_bf, obs_bf, lib, m2)
    return expressed, elbo


# speedup vs baseline: 4.1414x; 1.0046x over previous
import jax
import jax.numpy as jnp
from jax import lax
from jax.experimental import pallas as pl

N_DONORS = 64
N_CLUSTERS = 32
N_GENES = 2000
N_VARIANTS = 5000
N_VXG = 10000
EPS = 1e-8
_HALF_LOG_2PI = 0.9189385332046727

_BLK = 512


def _lgamma_pos(x):
    q = x * x + 3.0 * x
    p = q * (q + 2.0)
    small = x < 4.0
    z = jnp.where(small, x + 4.0, x)
    zi = 1.0 / z
    zi2 = zi * zi
    ser = zi * (0.08333333333333333 + zi2 * (-0.002777777777777778
                                             + zi2 * 0.0007936507936507937))
    st = (z - 0.5) * jnp.log(z) - z + _HALF_LOG_2PI + ser
    return jnp.where(small, st - jnp.log(p), st)


def _stirling(z):
    zi = 1.0 / z
    zi2 = zi * zi
    ser = zi * (0.08333333333333333 + zi2 * (-0.002777777777777778
                                             + zi2 * 0.0007936507936507937))
    return (z - 0.5) * jnp.log(z) - z + ser


def _lgamma_diff(xa, xb):
    qa = xa * xa + 3.0 * xa
    pa = qa * (qa + 2.0)
    small_a = xa < 4.0
    za = jnp.where(small_a, xa + 4.0, xa)
    qb = xb * xb + 3.0 * xb
    pb = qb * (qb + 2.0)
    small_b = xb < 4.0
    zb = jnp.where(small_b, xb + 4.0, xb)
    num = jnp.where(small_a, pa, 1.0)
    den = jnp.where(small_b, pb, 1.0)
    return _stirling(zb) - _stirling(za) + jnp.log(num / den)


def _prep_body(baseline_ref, dispersion_ref, m2_ref):
    disp = jnp.minimum(jnp.exp(dispersion_ref[...]), 20.0)
    tc = 1.0 / disp
    rows = (baseline_ref[...], tc, jnp.log(tc + EPS), _lgamma_pos(tc))
    for i, r in enumerate(rows):
        hi = r.astype(jnp.bfloat16)
        lo = (r - hi.astype(jnp.float32)).astype(jnp.bfloat16)
        m2_ref[i * N_CLUSTERS:(i + 1) * N_CLUSTERS, :] = hi
        m2_ref[(4 + i) * N_CLUSTERS:(5 + i) * N_CLUSTERS, :] = lo


def _main_body(gidx_ref, sidx_ref, lidx_ref, fc_ref, geno_ref, obs_ref,
               lib_ref, m2_ref, expressed_ref, elbo_ref):
    B = fc_ref.shape[-1]
    gidx = gidx_ref[0]
    sidx = sidx_ref[0]
    lidx = lidx_ref[0]

    iota_gene = lax.broadcasted_iota(jnp.int32, (N_GENES, B), 0)
    iota_var = lax.broadcasted_iota(jnp.int32, (N_VARIANTS, B), 0)
    oh_gene = (iota_gene == gidx).astype(jnp.bfloat16)
    oh_sel = (iota_var == sidx).astype(jnp.bfloat16)
    oh_lg = (iota_gene == lidx).astype(jnp.bfloat16)

    dn = (((1,), (0,)), ((), ()))
    mg2 = lax.dot_general(m2_ref[...], oh_gene, dn,
                          preferred_element_type=jnp.float32)
    mg = mg2[0:4 * N_CLUSTERS] + mg2[4 * N_CLUSTERS:]
    g = lax.dot_general(geno_ref[...], oh_sel, dn,
                        preferred_element_type=jnp.float32)
    value = lax.dot_general(obs_ref[...], oh_lg, dn,
                            preferred_element_type=jnp.float32)
    value = value.reshape(N_DONORS, N_CLUSTERS, B)

    baseline_g = mg[0:N_CLUSTERS]
    tc = mg[N_CLUSTERS:2 * N_CLUSTERS]
    l1 = mg[2 * N_CLUSTERS:3 * N_CLUSTERS]
    g0 = mg[3 * N_CLUSTERS:4 * N_CLUSTERS]

    el = baseline_g[None, :, :] + g[:, None, :] * fc_ref[...][None, :, :]
    expressed = jnp.exp(el) * lib_ref[...][:, :, None]
    expressed_ref[...] = expressed

    logits = jnp.log(expressed + EPS) - l1[None, :, :]
    sp = jnp.maximum(logits, 0.0) + jnp.log(1.0 + jnp.exp(-jnp.abs(logits)))
    tcv = tc[None, :, :] + value
    elbo = (tcv * sp - value * logits
            + _lgamma_diff(tcv, 1.0 + value) + g0[None, :, :])
    elbo_ref[...] = elbo


def kernel(fc_log, genotypes, expression_obs, variantxgene_to_gene,
           local_variant_to_local_variantxgene_selector, variantxgene_to_local_gene,
           lib, baseline_log, dispersion_log):
    nblk = (N_VXG + _BLK - 1) // _BLK
    pad = nblk * _BLK - N_VXG

    m2 = pl.pallas_call(
        _prep_body,
        out_shape=jax.ShapeDtypeStruct((8 * N_CLUSTERS, N_GENES), jnp.bfloat16),
    )(baseline_log, dispersion_log)

    def _idx(a):
        a = jnp.pad(a.astype(jnp.int32), (0, pad))
        return a.reshape(nblk, 1, _BLK)

    gidx = _idx(variantxgene_to_gene)
    sidx = _idx(local_variant_to_local_variantxgene_selector)
    lidx = _idx(variantxgene_to_local_gene)
    geno_bf = genotypes.astype(jnp.bfloat16)
    obs_bf = expression_obs.reshape(N_DONORS * N_CLUSTERS, N_GENES).astype(jnp.bfloat16)

    grid = (nblk,)
    out_specs = [
        pl.BlockSpec((N_DONORS, N_CLUSTERS, _BLK), lambda j: (0, 0, j)),
        pl.BlockSpec((N_DONORS, N_CLUSTERS, _BLK), lambda j: (0, 0, j)),
    ]
    in_specs = [
        pl.BlockSpec((1, 1, _BLK), lambda j: (j, 0, 0)),
        pl.BlockSpec((1, 1, _BLK), lambda j: (j, 0, 0)),
        pl.BlockSpec((1, 1, _BLK), lambda j: (j, 0, 0)),
        pl.BlockSpec((N_CLUSTERS, _BLK), lambda j: (0, j)),
        pl.BlockSpec((N_DONORS, N_VARIANTS), lambda j: (0, 0)),
        pl.BlockSpec((N_DONORS * N_CLUSTERS, N_GENES), lambda j: (0, 0)),
        pl.BlockSpec((N_DONORS, N_CLUSTERS), lambda j: (0, 0)),
        pl.BlockSpec((8 * N_CLUSTERS, N_GENES), lambda j: (0, 0)),
    ]
    expressed, elbo = pl.pallas_call(
        _main_body,
        grid=grid,
        in_specs=in_specs,
        out_specs=out_specs,
        out_shape=[
            jax.ShapeDtypeStruct((N_DONORS, N_CLUSTERS, N_VXG), jnp.float32),
            jax.ShapeDtypeStruct((N_DONORS, N_CLUSTERS, N_VXG), jnp.float32),
        ],
    )(gidx, sidx, lidx, fc_log, geno_bf, obs_bf, lib, m2)
    return expressed, elbo
